# baseline (speedup 1.0000x reference)
import functools

import jax
import jax.numpy as jnp
from jax import lax
from jax.experimental import pallas as pl
from jax.experimental.pallas import tpu as pltpu
from jax.experimental.pallas import tpu_sc as plsc

N = 10000
D = 128
E = 320000

_NC = 2
_NS = 16
_NW = _NC * _NS
_EPW = E // _NW
_CHUNK = 112
_NCHUNK = _EPW // _CHUNK
_TCH = _EPW - _NCHUNK * _CHUNK
_TOFF = _NCHUNK * _CHUNK
_NROW = 3
_NIB = 6
_RPS = 624
_TAIL = N - _NS * _RPS
_TAIL_OFF = _NS * _RPS


def _sc_agg_body(h_hbm, src_hbm, dst_hbm, zero_hbm, out_hbm, *sc):
    cid = lax.axis_index("c")
    sid = lax.axis_index("s")
    wid = sid * _NC + cid

    sibs = sc[0:_NIB]
    dibs = sc[_NIB:2 * _NIB]
    rows = sc[2 * _NIB:2 * _NIB + _NROW]
    tib_s, tib_d, trows = sc[2 * _NIB + _NROW:2 * _NIB + _NROW + 3]
    agg_sp = sc[2 * _NIB + _NROW + 3]
    o = 2 * _NIB + _NROW + 4
    isems_s = sc[o:o + _NIB]
    isems_d = sc[o + _NIB:o + 2 * _NIB]
    gsems = sc[o + 2 * _NIB:o + 2 * _NIB + _NROW]
    ssems = sc[o + 2 * _NIB + _NROW:o + 2 * _NIB + 2 * _NROW]
    tsem = sc[o + 2 * _NIB + 2 * _NROW]

    _GL = _NROW - 1
    _IL = _NIB - 1

    def base(t):
        return pl.multiple_of(wid * _EPW + t * _CHUNK, 16)

    def fire_i(t, b):
        pltpu.async_copy(src_hbm.at[pl.ds(base(t), _CHUNK)], sibs[b],
                         isems_s[b])
        pltpu.async_copy(dst_hbm.at[pl.ds(base(t), _CHUNK)], dibs[b],
                         isems_d[b])

    def wait_i(t, b):
        pltpu.make_async_copy(src_hbm.at[pl.ds(base(t), _CHUNK)], sibs[b],
                              isems_s[b]).wait()
        pltpu.make_async_copy(dst_hbm.at[pl.ds(base(t), _CHUNK)], dibs[b],
                              isems_d[b]).wait()

    def fire_g(b):
        pltpu.async_copy(h_hbm.at[sibs[b]], rows[b % _NROW],
                         gsems[b % _NROW])

    def wait_g(b):
        pltpu.make_async_copy(h_hbm.at[sibs[b]], rows[b % _NROW],
                              gsems[b % _NROW]).wait()

    def fire_s(b):
        pltpu.async_copy(rows[b % _NROW], agg_sp.at[dibs[b]],
                         ssems[b % _NROW], add=True)

    def wait_s(b):
        pltpu.make_async_copy(rows[b % _NROW], agg_sp.at[dibs[b]],
                              ssems[b % _NROW]).wait()

    def step(t, b, first=False, fi=True, fg=True):
        if not first:
            wait_s((b - 1) % _NIB)
        if fi:
            fire_i(t + _IL, (b + _IL) % _NIB)
        if fg:
            wait_i(t + _GL, (b + _GL) % _NIB)
            fire_g((b + _GL) % _NIB)
        wait_g(b)
        fire_s(b)

    tb = pl.multiple_of(wid * _EPW + _TOFF, 16)
    pltpu.async_copy(src_hbm.at[pl.ds(tb, _TCH)], tib_s, tsem)
    pltpu.async_copy(dst_hbm.at[pl.ds(tb, _TCH)], tib_d, tsem)
    for t in range(_IL):
        fire_i(t, t)
    pltpu.make_async_copy(src_hbm.at[pl.ds(tb, _TCH)], tib_s, tsem).wait()
    pltpu.make_async_copy(dst_hbm.at[pl.ds(tb, _TCH)], tib_d, tsem).wait()
    tg = pltpu.async_copy(h_hbm.at[tib_s], trows, tsem)
    for t in range(_GL):
        wait_i(t, t)
        fire_g(t)

    pltpu.sync_copy(zero_hbm.at[pl.ds(0, _RPS)],
                    agg_sp.at[pl.ds(sid * _RPS, _RPS)])

    @pl.when(sid == 0)
    def _():
        pltpu.sync_copy(zero_hbm.at[pl.ds(0, _TAIL)],
                        agg_sp.at[pl.ds(_TAIL_OFF, _TAIL)])

    plsc.subcore_barrier()

    tg.wait()
    pltpu.async_copy(trows, agg_sp.at[tib_d], tsem, add=True)

    for t in range(_NIB):
        step(t, t, first=(t == 0))

    def body(k, carry):
        t0 = _NIB * k
        for j in range(_NIB):
            step(t0 + j, j)
        return carry

    lax.fori_loop(1, _NCHUNK // _NIB, body, 0)

    for t in range(_NIB * (_NCHUNK // _NIB), _NCHUNK):
        step(t, t % _NIB, fi=(t + _IL < _NCHUNK), fg=(t + _GL < _NCHUNK))
    wait_s((_NCHUNK - 1) % _NIB)
    pltpu.make_async_copy(trows, agg_sp.at[tib_d], tsem).wait()
    plsc.subcore_barrier()
    pltpu.sync_copy(agg_sp.at[pl.ds(sid * _RPS, _RPS)],
                    out_hbm.at[cid, pl.ds(sid * _RPS, _RPS)])

    @pl.when(sid == 0)
    def _():
        pltpu.sync_copy(agg_sp.at[pl.ds(_TAIL_OFF, _TAIL)],
                        out_hbm.at[cid, pl.ds(_TAIL_OFF, _TAIL)])


def _sc_agg(h, src, dst, zeros):
    mesh = plsc.VectorSubcoreMesh(core_axis_name="c", subcore_axis_name="s")
    f = pl.kernel(
        _sc_agg_body,
        mesh=mesh,
        out_type=jax.ShapeDtypeStruct((_NC, N, D), jnp.float32),
        scratch_types=(
            [pltpu.VMEM((_CHUNK,), jnp.int32)] * (2 * _NIB)
            + [pltpu.VMEM((_CHUNK, D), jnp.float32)] * _NROW
            + [pltpu.VMEM((_TCH,), jnp.int32)] * 2
            + [pltpu.VMEM((_TCH, D), jnp.float32)]
            + [pltpu.VMEM_SHARED((N, D), jnp.float32)]
            + [pltpu.SemaphoreType.DMA] * (2 * _NIB + 2 * _NROW + 1)
        ),
    )
    return f(h, src, dst, zeros)


_BLK = 2000


def _mlp_body(h_ref, p_ref, w1_ref, b1_ref, w2_ref, b2_ref, o_ref):
    hv = h_ref[...] + p_ref[0] + p_ref[1]
    z = jnp.dot(hv, w1_ref[...], preferred_element_type=jnp.float32)
    z = jnp.maximum(z + b1_ref[...], 0.0)
    z = jnp.dot(z, w2_ref[...], preferred_element_type=jnp.float32)
    o_ref[...] = jnp.maximum(z + b2_ref[...], 0.0)


def _tc_mlp(h, p, w1, b1, w2, b2):
    return pl.pallas_call(
        _mlp_body,
        grid=(N // _BLK,),
        in_specs=[
            pl.BlockSpec((_BLK, D), lambda i: (i, 0)),
            pl.BlockSpec((_NC, _BLK, D), lambda i: (0, i, 0)),
            pl.BlockSpec((D, D), lambda i: (0, 0)),
            pl.BlockSpec((1, D), lambda i: (0, 0)),
            pl.BlockSpec((D, D), lambda i: (0, 0)),
            pl.BlockSpec((1, D), lambda i: (0, 0)),
        ],
        out_specs=pl.BlockSpec((_BLK, D), lambda i: (i, 0)),
        out_shape=jax.ShapeDtypeStruct((N, D), jnp.float32),
    )(h, p, w1, b1.reshape(1, D), w2, b2.reshape(1, D))


def _final_body(h_ref, p_ref, w1_ref, b1_ref, w2_ref, b2_ref,
                fw1_ref, fb1_ref, fw2_ref, fb2_ref, o_ref):
    hv = h_ref[...] + p_ref[0] + p_ref[1]
    z = jnp.dot(hv, w1_ref[...], preferred_element_type=jnp.float32)
    z = jnp.maximum(z + b1_ref[...], 0.0)
    z = jnp.dot(z, w2_ref[...], preferred_element_type=jnp.float32)
    z = jnp.maximum(z + b2_ref[...], 0.0)
    z = jnp.dot(z, fw1_ref[...], preferred_element_type=jnp.float32)
    z = jnp.maximum(z + fb1_ref[...], 0.0)
    z = jnp.dot(z, fw2_ref[...], preferred_element_type=jnp.float32)
    o_ref[...] = z + fb2_ref[...]


def _tc_final(h, p, w1, b1, w2, b2, fw1, fb1, fw2, fb2):
    wspec = pl.BlockSpec((D, D), lambda i: (0, 0))
    bspec = pl.BlockSpec((1, D), lambda i: (0, 0))
    return pl.pallas_call(
        _final_body,
        grid=(N // _BLK,),
        in_specs=[
            pl.BlockSpec((_BLK, D), lambda i: (i, 0)),
            pl.BlockSpec((_NC, _BLK, D), lambda i: (0, i, 0)),
            wspec, bspec, wspec, bspec, wspec, bspec, wspec, bspec,
        ],
        out_specs=pl.BlockSpec((_BLK, D), lambda i: (i, 0)),
        out_shape=jax.ShapeDtypeStruct((N, D), jnp.float32),
    )(h, p, w1, b1.reshape(1, D), w2, b2.reshape(1, D),
      fw1, fb1.reshape(1, D), fw2, fb2.reshape(1, D))


def kernel(x, edge_index, c1w1, c1b1, c1w2, c1b2, c2w1, c2b1, c2w2, c2b2,
           c3w1, c3b1, c3w2, c3b2, fcw1, fcb1, fcw2, fcb2):
    src = edge_index[0]
    dst = edge_index[1]
    zeros = jnp.zeros((_RPS, D), jnp.float32)
    p = _sc_agg(x, src, dst, zeros)
    h = _tc_mlp(x, p, c1w1, c1b1, c1w2, c1b2)
    p = _sc_agg(h, src, dst, zeros)
    h = _tc_mlp(h, p, c2w1, c2b1, c2w2, c2b2)
    p = _sc_agg(h, src, dst, zeros)
    return _tc_final(h, p, c3w1, c3b1, c3w2, c3b2, fcw1, fcb1, fcw2, fcb2)


# speedup vs baseline: 1.0023x; 1.0023x over previous
import functools

import jax
import jax.numpy as jnp
from jax import lax
from jax.experimental import pallas as pl
from jax.experimental.pallas import tpu as pltpu
from jax.experimental.pallas import tpu_sc as plsc

N = 10000
D = 128
E = 320000

_NC = 2
_NS = 16
_NW = _NC * _NS
_EPW = E // _NW
_CHUNK = 80
_NCHUNK = _EPW // _CHUNK
_TCH = _EPW - _NCHUNK * _CHUNK
_TOFF = _NCHUNK * _CHUNK
_NROW = 4
_NIB = 8
_IL = 5
_RPS = 624
_TAIL = N - _NS * _RPS
_TAIL_OFF = _NS * _RPS


def _sc_agg_body(h_hbm, src_hbm, dst_hbm, zero_hbm, out_hbm, *sc):
    cid = lax.axis_index("c")
    sid = lax.axis_index("s")
    wid = sid * _NC + cid

    nt = 3 if _TCH else 0
    sibs = sc[0:_NIB]
    dibs = sc[_NIB:2 * _NIB]
    rows = sc[2 * _NIB:2 * _NIB + _NROW]
    if _TCH:
        tib_s, tib_d, trows = sc[2 * _NIB + _NROW:2 * _NIB + _NROW + 3]
    agg_sp = sc[2 * _NIB + _NROW + nt]
    o = 2 * _NIB + _NROW + nt + 1
    isems_s = sc[o:o + _NIB]
    isems_d = sc[o + _NIB:o + 2 * _NIB]
    gsems = sc[o + 2 * _NIB:o + 2 * _NIB + _NROW]
    ssems = sc[o + 2 * _NIB + _NROW:o + 2 * _NIB + 2 * _NROW]
    if _TCH:
        tsem = sc[o + 2 * _NIB + 2 * _NROW]

    _GL = _NROW - 1

    def base(t):
        return pl.multiple_of(wid * _EPW + t * _CHUNK, 16)

    def fire_i(t, b):
        pltpu.async_copy(src_hbm.at[pl.ds(base(t), _CHUNK)], sibs[b],
                         isems_s[b])
        pltpu.async_copy(dst_hbm.at[pl.ds(base(t), _CHUNK)], dibs[b],
                         isems_d[b])

    def wait_i(t, b):
        pltpu.make_async_copy(src_hbm.at[pl.ds(base(t), _CHUNK)], sibs[b],
                              isems_s[b]).wait()
        pltpu.make_async_copy(dst_hbm.at[pl.ds(base(t), _CHUNK)], dibs[b],
                              isems_d[b]).wait()

    def fire_g(b):
        pltpu.async_copy(h_hbm.at[sibs[b]], rows[b % _NROW],
                         gsems[b % _NROW])

    def wait_g(b):
        pltpu.make_async_copy(h_hbm.at[sibs[b]], rows[b % _NROW],
                              gsems[b % _NROW]).wait()

    def fire_s(b):
        pltpu.async_copy(rows[b % _NROW], agg_sp.at[dibs[b]],
                         ssems[b % _NROW], add=True)

    def wait_s(b):
        pltpu.make_async_copy(rows[b % _NROW], agg_sp.at[dibs[b]],
                              ssems[b % _NROW]).wait()

    def step(t, b, first=False, fi=True, fg=True):
        if not first:
            wait_s((b - 1) % _NIB)
        if fi:
            fire_i(t + _IL, (b + _IL) % _NIB)
        if fg:
            wait_i(t + _GL, (b + _GL) % _NIB)
            fire_g((b + _GL) % _NIB)
        wait_g(b)
        fire_s(b)

    if _TCH:
        tb = pl.multiple_of(wid * _EPW + _TOFF, 16)
        pltpu.async_copy(src_hbm.at[pl.ds(tb, _TCH)], tib_s, tsem)
        pltpu.async_copy(dst_hbm.at[pl.ds(tb, _TCH)], tib_d, tsem)
    for t in range(_IL):
        fire_i(t, t)
    if _TCH:
        pltpu.make_async_copy(src_hbm.at[pl.ds(tb, _TCH)], tib_s,
                              tsem).wait()
        pltpu.make_async_copy(dst_hbm.at[pl.ds(tb, _TCH)], tib_d,
                              tsem).wait()
        tg = pltpu.async_copy(h_hbm.at[tib_s], trows, tsem)
    for t in range(_GL):
        wait_i(t, t)
        fire_g(t)

    pltpu.sync_copy(zero_hbm.at[pl.ds(0, _RPS)],
                    agg_sp.at[pl.ds(sid * _RPS, _RPS)])

    @pl.when(sid == 0)
    def _():
        pltpu.sync_copy(zero_hbm.at[pl.ds(0, _TAIL)],
                        agg_sp.at[pl.ds(_TAIL_OFF, _TAIL)])

    plsc.subcore_barrier()

    if _TCH:
        tg.wait()
        pltpu.async_copy(trows, agg_sp.at[tib_d], tsem, add=True)

    for t in range(_NIB):
        step(t, t, first=(t == 0))

    def body(k, carry):
        t0 = _NIB * k
        for j in range(_NIB):
            step(t0 + j, j)
        return carry

    lax.fori_loop(1, _NCHUNK // _NIB, body, 0)

    for t in range(_NIB * (_NCHUNK // _NIB), _NCHUNK):
        step(t, t % _NIB, fi=(t + _IL < _NCHUNK), fg=(t + _GL < _NCHUNK))
    wait_s((_NCHUNK - 1) % _NIB)
    if _TCH:
        pltpu.make_async_copy(trows, agg_sp.at[tib_d], tsem).wait()
    plsc.subcore_barrier()
    pltpu.sync_copy(agg_sp.at[pl.ds(sid * _RPS, _RPS)],
                    out_hbm.at[cid, pl.ds(sid * _RPS, _RPS)])

    @pl.when(sid == 0)
    def _():
        pltpu.sync_copy(agg_sp.at[pl.ds(_TAIL_OFF, _TAIL)],
                        out_hbm.at[cid, pl.ds(_TAIL_OFF, _TAIL)])


def _sc_agg(h, src, dst, zeros):
    mesh = plsc.VectorSubcoreMesh(core_axis_name="c", subcore_axis_name="s")
    f = pl.kernel(
        _sc_agg_body,
        mesh=mesh,
        out_type=jax.ShapeDtypeStruct((_NC, N, D), jnp.float32),
        scratch_types=(
            [pltpu.VMEM((_CHUNK,), jnp.int32)] * (2 * _NIB)
            + [pltpu.VMEM((_CHUNK, D), jnp.float32)] * _NROW
            + ([pltpu.VMEM((_TCH,), jnp.int32)] * 2
               + [pltpu.VMEM((_TCH, D), jnp.float32)] if _TCH else [])
            + [pltpu.VMEM_SHARED((N, D), jnp.float32)]
            + [pltpu.SemaphoreType.DMA] * (2 * _NIB + 2 * _NROW
                                           + (1 if _TCH else 0))
        ),
    )
    return f(h, src, dst, zeros)


_BLK = 2000


def _mlp_body(h_ref, p_ref, w1_ref, b1_ref, w2_ref, b2_ref, o_ref):
    hv = h_ref[...] + p_ref[0] + p_ref[1]
    z = jnp.dot(hv, w1_ref[...], preferred_element_type=jnp.float32)
    z = jnp.maximum(z + b1_ref[...], 0.0)
    z = jnp.dot(z, w2_ref[...], preferred_element_type=jnp.float32)
    o_ref[...] = jnp.maximum(z + b2_ref[...], 0.0)


def _tc_mlp(h, p, w1, b1, w2, b2):
    return pl.pallas_call(
        _mlp_body,
        grid=(N // _BLK,),
        in_specs=[
            pl.BlockSpec((_BLK, D), lambda i: (i, 0)),
            pl.BlockSpec((_NC, _BLK, D), lambda i: (0, i, 0)),
            pl.BlockSpec((D, D), lambda i: (0, 0)),
            pl.BlockSpec((1, D), lambda i: (0, 0)),
            pl.BlockSpec((D, D), lambda i: (0, 0)),
            pl.BlockSpec((1, D), lambda i: (0, 0)),
        ],
        out_specs=pl.BlockSpec((_BLK, D), lambda i: (i, 0)),
        out_shape=jax.ShapeDtypeStruct((N, D), jnp.float32),
    )(h, p, w1, b1.reshape(1, D), w2, b2.reshape(1, D))


def _final_body(h_ref, p_ref, w1_ref, b1_ref, w2_ref, b2_ref,
                fw1_ref, fb1_ref, fw2_ref, fb2_ref, o_ref):
    hv = h_ref[...] + p_ref[0] + p_ref[1]
    z = jnp.dot(hv, w1_ref[...], preferred_element_type=jnp.float32)
    z = jnp.maximum(z + b1_ref[...], 0.0)
    z = jnp.dot(z, w2_ref[...], preferred_element_type=jnp.float32)
    z = jnp.maximum(z + b2_ref[...], 0.0)
    z = jnp.dot(z, fw1_ref[...], preferred_element_type=jnp.float32)
    z = jnp.maximum(z + fb1_ref[...], 0.0)
    z = jnp.dot(z, fw2_ref[...], preferred_element_type=jnp.float32)
    o_ref[...] = z + fb2_ref[...]


def _tc_final(h, p, w1, b1, w2, b2, fw1, fb1, fw2, fb2):
    wspec = pl.BlockSpec((D, D), lambda i: (0, 0))
    bspec = pl.BlockSpec((1, D), lambda i: (0, 0))
    return pl.pallas_call(
        _final_body,
        grid=(N // _BLK,),
        in_specs=[
            pl.BlockSpec((_BLK, D), lambda i: (i, 0)),
            pl.BlockSpec((_NC, _BLK, D), lambda i: (0, i, 0)),
            wspec, bspec, wspec, bspec, wspec, bspec, wspec, bspec,
        ],
        out_specs=pl.BlockSpec((_BLK, D), lambda i: (i, 0)),
        out_shape=jax.ShapeDtypeStruct((N, D), jnp.float32),
    )(h, p, w1, b1.reshape(1, D), w2, b2.reshape(1, D),
      fw1, fb1.reshape(1, D), fw2, fb2.reshape(1, D))


def kernel(x, edge_index, c1w1, c1b1, c1w2, c1b2, c2w1, c2b1, c2w2, c2b2,
           c3w1, c3b1, c3w2, c3b2, fcw1, fcb1, fcw2, fcb2):
    src = edge_index[0]
    dst = edge_index[1]
    zeros = jnp.zeros((_RPS, D), jnp.float32)
    p = _sc_agg(x, src, dst, zeros)
    h = _tc_mlp(x, p, c1w1, c1b1, c1w2, c1b2)
    p = _sc_agg(h, src, dst, zeros)
    h = _tc_mlp(h, p, c2w1, c2b1, c2w2, c2b2)
    p = _sc_agg(h, src, dst, zeros)
    return _tc_final(h, p, c3w1, c3b1, c3w2, c3b2, fcw1, fcb1, fcw2, fcb2)
